# baseline (speedup 1.0000x reference)
import jax
import jax.numpy as jnp
from jax import lax
from jax.experimental import pallas as pl
from jax.experimental.pallas import tpu as pltpu
from jax.experimental.pallas import tpu_sc as plsc

_NC = 2
_NS = 16
_NW = _NC * _NS
_L = 16
_CHUNK = 128


def _sc_line_gather_body(lidx_hbm, emb2_hbm, out_hbm, lidx_v, lines_v, sem):
    k = lidx_v.shape[0]
    bpw = k * _CHUNK
    wid = lax.axis_index("s") * _NC + lax.axis_index("c")
    base = pl.multiple_of(wid * bpw, bpw)
    pltpu.sync_copy(lidx_hbm.at[wid], lidx_v)
    copies = []
    for j in range(k):
        copies.append(pltpu.async_copy(
            emb2_hbm.at[lidx_v.at[j]],
            lines_v.at[pl.ds(j * _CHUNK, _CHUNK)], sem))
    for c in copies:
        c.wait()
    pltpu.sync_copy(lines_v, out_hbm.at[pl.ds(base, bpw)])


def _sc_bias_gather_body(uidx_hbm, pidx_hbm, uridx_hbm, pridx_hbm,
                         ubias16_hbm, pbias16_hbm, ub_out, pb_out,
                         uidx_v, pidx_v, uridx_v, pridx_v,
                         ubrows_v, pbrows_v, ubvals_v, pbvals_v, sem):
    k = uidx_v.shape[0]
    chunk = uidx_v.shape[1]
    bpw = k * chunk
    wid = lax.axis_index("s") * _NC + lax.axis_index("c")
    base = wid * bpw
    pltpu.sync_copy(uidx_hbm.at[wid], uidx_v)
    pltpu.sync_copy(pidx_hbm.at[wid], pidx_v)
    pltpu.sync_copy(uridx_hbm.at[wid], uridx_v)
    pltpu.sync_copy(pridx_hbm.at[wid], pridx_v)
    copies = []
    for j in range(k):
        sl = pl.ds(j * chunk, chunk)
        copies.append(pltpu.async_copy(
            ubias16_hbm.at[uridx_v.at[j]], ubrows_v.at[sl], sem))
        copies.append(pltpu.async_copy(
            pbias16_hbm.at[pridx_v.at[j]], pbrows_v.at[sl], sem))
    for c in copies:
        c.wait()
    lane_iota = lax.iota(jnp.int32, _L)
    for j in range(k):
        for c in range(chunk // _L):
            off = j * chunk + c * _L
            jvec = off + lane_iota
            usl = uidx_v.at[j][pl.ds(c * _L, _L)] & (_L - 1)
            psl = pidx_v.at[j][pl.ds(c * _L, _L)] & (_L - 1)
            ubvals_v[pl.ds(off, _L)] = plsc.load_gather(ubrows_v, [jvec, usl])
            pbvals_v[pl.ds(off, _L)] = plsc.load_gather(pbrows_v, [jvec, psl])
    pltpu.sync_copy(ubvals_v, ub_out.at[pl.ds(base, bpw)])
    pltpu.sync_copy(pbvals_v, pb_out.at[pl.ds(base, bpw)])


_PBLK = 16384
_PSH = 14


def _tc_pack_body(in_ref, out_ref):
    t = jnp.transpose(in_ref[...])
    h = t.shape[0] // 2
    out_ref[...] = jnp.concatenate([t[:h], t[h:]], axis=1)


def _pack_rows(embT):
    F, N = embT.shape
    nblk = pl.cdiv(N, _PBLK)
    hb = _PBLK // 2
    return pl.pallas_call(
        _tc_pack_body,
        grid=(nblk,),
        in_specs=[pl.BlockSpec((F, _PBLK), lambda i: (0, i))],
        out_specs=pl.BlockSpec((hb, 2 * F), lambda i: (i, 0)),
        out_shape=jax.ShapeDtypeStruct((nblk * hb, 2 * F), jnp.float32),
    )(embT)


def _tc_mlp_body(ul_ref, pl_ref, uodd_ref, podd_ref, ub_ref, pb_ref,
                 w1u_ref, w1p_ref, b1_ref, w2_ref, b2_ref, w3_ref,
                 b3gb_ref, out_ref):
    f = w1u_ref.shape[0]
    ul = ul_ref[...]
    pll = pl_ref[...]
    u = jnp.where(uodd_ref[...] > 0, ul[:, f:], ul[:, :f])
    p = jnp.where(podd_ref[...] > 0, pll[:, f:], pll[:, :f])
    dot = jnp.sum(u * p, axis=1, keepdims=True)
    h = jnp.dot(u, w1u_ref[...], preferred_element_type=jnp.float32)
    h = h + jnp.dot(p, w1p_ref[...], preferred_element_type=jnp.float32)
    h = jnp.maximum(h + b1_ref[...], 0.0)
    h = jnp.maximum(
        jnp.dot(h, w2_ref[...], preferred_element_type=jnp.float32)
        + b2_ref[...], 0.0)
    mlp = jnp.sum(h * w3_ref[...], axis=1, keepdims=True)
    out_ref[...] = (dot + mlp + ub_ref[...] + pb_ref[...] + b3gb_ref[...])


def kernel(user_idx, prob_idx, user_emb, prob_emb, user_bias, prob_bias,
           global_bias, W1, b1, W2, b2, W3, b3):
    B = user_idx.shape[0]
    F = user_emb.shape[1]
    H1 = W1.shape[0]
    H2 = W2.shape[0]
    bpw = B // _NW
    k = bpw // _CHUNK

    uidx = user_idx.astype(jnp.int32)
    pidx = prob_idx.astype(jnp.int32)
    hb = _PBLK // 2
    ulidx = ((uidx >> _PSH) << (_PSH - 1)) | (uidx & (hb - 1))
    plidx = ((pidx >> _PSH) << (_PSH - 1)) | (pidx & (hb - 1))
    ulidx3 = ulidx.reshape(_NW, k, _CHUNK)
    plidx3 = plidx.reshape(_NW, k, _CHUNK)

    line_call = pl.kernel(
        _sc_line_gather_body,
        out_type=jax.ShapeDtypeStruct((B, 2 * F), jnp.float32),
        mesh=plsc.VectorSubcoreMesh(core_axis_name="c", subcore_axis_name="s"),
        scratch_types=[
            pltpu.VMEM((k, _CHUNK), jnp.int32),
            pltpu.VMEM((bpw, 2 * F), jnp.float32),
            pltpu.SemaphoreType.DMA,
        ],
    )
    pemb2 = _pack_rows(prob_emb.T)
    p_lines = line_call(plidx3, pemb2)
    uemb2 = _pack_rows(user_emb.T)
    u_lines = line_call(ulidx3, uemb2)

    uidx3 = uidx.reshape(_NW, k, _CHUNK)
    pidx3 = pidx.reshape(_NW, k, _CHUNK)
    uridx3 = (uidx >> 4).reshape(_NW, k, _CHUNK)
    pridx3 = (pidx >> 4).reshape(_NW, k, _CHUNK)
    ubias16 = user_bias.reshape(-1, _L)
    pbias16 = prob_bias.reshape(-1, _L)

    bias_call = pl.kernel(
        _sc_bias_gather_body,
        out_type=[
            jax.ShapeDtypeStruct((B,), jnp.float32),
            jax.ShapeDtypeStruct((B,), jnp.float32),
        ],
        mesh=plsc.VectorSubcoreMesh(core_axis_name="c", subcore_axis_name="s"),
        scratch_types=[
            pltpu.VMEM((k, _CHUNK), jnp.int32),
            pltpu.VMEM((k, _CHUNK), jnp.int32),
            pltpu.VMEM((k, _CHUNK), jnp.int32),
            pltpu.VMEM((k, _CHUNK), jnp.int32),
            pltpu.VMEM((bpw, _L), jnp.float32),
            pltpu.VMEM((bpw, _L), jnp.float32),
            pltpu.VMEM((bpw,), jnp.float32),
            pltpu.VMEM((bpw,), jnp.float32),
            pltpu.SemaphoreType.DMA,
        ],
        compiler_params=pltpu.CompilerParams(
            use_tc_tiling_on_sc=False, needs_layout_passes=False),
    )
    ub, pb = bias_call(uidx3, pidx3, uridx3, pridx3, ubias16, pbias16)

    uodd = ((uidx >> (_PSH - 1)) & 1).reshape(B, 1)
    podd = ((pidx >> (_PSH - 1)) & 1).reshape(B, 1)

    w1u = W1[:, :F].T
    w1p = W1[:, F:].T
    w2t = W2.T
    b1r = b1.reshape(1, H1)
    b2r = b2.reshape(1, H2)
    b3gb = (b3 + global_bias).reshape(1, 1)

    blk = 2048
    out = pl.pallas_call(
        _tc_mlp_body,
        grid=(B // blk,),
        in_specs=[
            pl.BlockSpec((blk, 2 * F), lambda i: (i, 0)),
            pl.BlockSpec((blk, 2 * F), lambda i: (i, 0)),
            pl.BlockSpec((blk, 1), lambda i: (i, 0)),
            pl.BlockSpec((blk, 1), lambda i: (i, 0)),
            pl.BlockSpec((blk, 1), lambda i: (i, 0)),
            pl.BlockSpec((blk, 1), lambda i: (i, 0)),
            pl.BlockSpec((F, H1), lambda i: (0, 0)),
            pl.BlockSpec((F, H1), lambda i: (0, 0)),
            pl.BlockSpec((1, H1), lambda i: (0, 0)),
            pl.BlockSpec((H1, H2), lambda i: (0, 0)),
            pl.BlockSpec((1, H2), lambda i: (0, 0)),
            pl.BlockSpec((1, H2), lambda i: (0, 0)),
            pl.BlockSpec((1, 1), lambda i: (0, 0)),
        ],
        out_specs=pl.BlockSpec((blk, 1), lambda i: (i, 0)),
        out_shape=jax.ShapeDtypeStruct((B, 1), jnp.float32),
    )(u_lines, p_lines, uodd, podd, ub.reshape(B, 1), pb.reshape(B, 1),
      w1u, w1p, b1r, w2t, b2r, W3, b3gb)
    return out[:, 0]


# speedup vs baseline: 1.0301x; 1.0301x over previous
import jax
import jax.numpy as jnp
from jax import lax
from jax.experimental import pallas as pl
from jax.experimental.pallas import tpu as pltpu
from jax.experimental.pallas import tpu_sc as plsc

_NC = 2
_NS = 16
_NW = _NC * _NS
_L = 16
_CHUNK = 128


def _sc_line_gather_body(lidx_hbm, emb2_hbm, out_hbm, lidx_v, lines_v, sem):
    k = lidx_v.shape[0]
    bpw = k * _CHUNK
    wid = lax.axis_index("s") * _NC + lax.axis_index("c")
    base = pl.multiple_of(wid * bpw, bpw)
    pltpu.sync_copy(lidx_hbm.at[wid], lidx_v)
    copies = []
    for j in range(k):
        copies.append(pltpu.async_copy(
            emb2_hbm.at[lidx_v.at[j]],
            lines_v.at[pl.ds(j * _CHUNK, _CHUNK)], sem))
    for c in copies:
        c.wait()
    pltpu.sync_copy(lines_v, out_hbm.at[pl.ds(base, bpw)])


def _sc_bias_gather_body(uidx_hbm, pidx_hbm, uridx_hbm, pridx_hbm,
                         ubias16_hbm, pbias16_hbm, ub_out, pb_out,
                         uidx_v, pidx_v, uridx_v, pridx_v,
                         ubrows_v, pbrows_v, ubvals_v, pbvals_v, sem):
    k = uidx_v.shape[0]
    chunk = uidx_v.shape[1]
    bpw = k * chunk
    wid = lax.axis_index("s") * _NC + lax.axis_index("c")
    base = wid * bpw
    pltpu.sync_copy(uidx_hbm.at[wid], uidx_v)
    pltpu.sync_copy(pidx_hbm.at[wid], pidx_v)
    pltpu.sync_copy(uridx_hbm.at[wid], uridx_v)
    pltpu.sync_copy(pridx_hbm.at[wid], pridx_v)
    copies = []
    for j in range(k):
        sl = pl.ds(j * chunk, chunk)
        copies.append(pltpu.async_copy(
            ubias16_hbm.at[uridx_v.at[j]], ubrows_v.at[sl], sem))
        copies.append(pltpu.async_copy(
            pbias16_hbm.at[pridx_v.at[j]], pbrows_v.at[sl], sem))
    for c in copies:
        c.wait()
    lane_iota = lax.iota(jnp.int32, _L)
    for j in range(k):
        for c in range(chunk // _L):
            off = j * chunk + c * _L
            jvec = off + lane_iota
            usl = uidx_v.at[j][pl.ds(c * _L, _L)] & (_L - 1)
            psl = pidx_v.at[j][pl.ds(c * _L, _L)] & (_L - 1)
            ubvals_v[pl.ds(off, _L)] = plsc.load_gather(ubrows_v, [jvec, usl])
            pbvals_v[pl.ds(off, _L)] = plsc.load_gather(pbrows_v, [jvec, psl])
    pltpu.sync_copy(ubvals_v, ub_out.at[pl.ds(base, bpw)])
    pltpu.sync_copy(pbvals_v, pb_out.at[pl.ds(base, bpw)])


_PBLK = 32768
_PSH = 15


def _tc_pack_body(in_ref, out_ref):
    t = jnp.transpose(in_ref[...])
    h = t.shape[0] // 2
    out_ref[...] = jnp.concatenate([t[:h], t[h:]], axis=1)


def _pack_rows(embT):
    F, N = embT.shape
    nblk = pl.cdiv(N, _PBLK)
    hb = _PBLK // 2
    return pl.pallas_call(
        _tc_pack_body,
        grid=(nblk,),
        in_specs=[pl.BlockSpec((F, _PBLK), lambda i: (0, i))],
        out_specs=pl.BlockSpec((hb, 2 * F), lambda i: (i, 0)),
        out_shape=jax.ShapeDtypeStruct((nblk * hb, 2 * F), jnp.float32),
    )(embT)


def _tc_mlp_body(ul_ref, pl_ref, uodd_ref, podd_ref, ub_ref, pb_ref,
                 w1u_ref, w1p_ref, b1_ref, w2_ref, b2_ref, w3_ref,
                 b3gb_ref, out_ref):
    f = w1u_ref.shape[0]
    ul = ul_ref[...]
    pll = pl_ref[...]
    u = jnp.where(uodd_ref[...] > 0, ul[:, f:], ul[:, :f])
    p = jnp.where(podd_ref[...] > 0, pll[:, f:], pll[:, :f])
    dot = jnp.sum(u * p, axis=1, keepdims=True)
    h = jnp.dot(u, w1u_ref[...], preferred_element_type=jnp.float32)
    h = h + jnp.dot(p, w1p_ref[...], preferred_element_type=jnp.float32)
    h = jnp.maximum(h + b1_ref[...], 0.0)
    h = jnp.maximum(
        jnp.dot(h, w2_ref[...], preferred_element_type=jnp.float32)
        + b2_ref[...], 0.0)
    mlp = jnp.sum(h * w3_ref[...], axis=1, keepdims=True)
    out_ref[...] = (dot + mlp + ub_ref[...] + pb_ref[...] + b3gb_ref[...])


def kernel(user_idx, prob_idx, user_emb, prob_emb, user_bias, prob_bias,
           global_bias, W1, b1, W2, b2, W3, b3):
    B = user_idx.shape[0]
    F = user_emb.shape[1]
    H1 = W1.shape[0]
    H2 = W2.shape[0]
    bpw = B // _NW
    k = bpw // _CHUNK

    uidx = user_idx.astype(jnp.int32)
    pidx = prob_idx.astype(jnp.int32)
    hb = _PBLK // 2
    ulidx = ((uidx >> _PSH) << (_PSH - 1)) | (uidx & (hb - 1))
    plidx = ((pidx >> _PSH) << (_PSH - 1)) | (pidx & (hb - 1))
    ulidx3 = ulidx.reshape(_NW, k, _CHUNK)
    plidx3 = plidx.reshape(_NW, k, _CHUNK)

    line_call = pl.kernel(
        _sc_line_gather_body,
        out_type=jax.ShapeDtypeStruct((B, 2 * F), jnp.float32),
        mesh=plsc.VectorSubcoreMesh(core_axis_name="c", subcore_axis_name="s"),
        scratch_types=[
            pltpu.VMEM((k, _CHUNK), jnp.int32),
            pltpu.VMEM((bpw, 2 * F), jnp.float32),
            pltpu.SemaphoreType.DMA,
        ],
    )
    pemb2 = _pack_rows(prob_emb.T)
    p_lines = line_call(plidx3, pemb2)
    uemb2 = _pack_rows(user_emb.T)
    u_lines = line_call(ulidx3, uemb2)

    uidx3 = uidx.reshape(_NW, k, _CHUNK)
    pidx3 = pidx.reshape(_NW, k, _CHUNK)
    uridx3 = (uidx >> 4).reshape(_NW, k, _CHUNK)
    pridx3 = (pidx >> 4).reshape(_NW, k, _CHUNK)
    ubias16 = user_bias.reshape(-1, _L)
    pbias16 = prob_bias.reshape(-1, _L)

    bias_call = pl.kernel(
        _sc_bias_gather_body,
        out_type=[
            jax.ShapeDtypeStruct((B,), jnp.float32),
            jax.ShapeDtypeStruct((B,), jnp.float32),
        ],
        mesh=plsc.VectorSubcoreMesh(core_axis_name="c", subcore_axis_name="s"),
        scratch_types=[
            pltpu.VMEM((k, _CHUNK), jnp.int32),
            pltpu.VMEM((k, _CHUNK), jnp.int32),
            pltpu.VMEM((k, _CHUNK), jnp.int32),
            pltpu.VMEM((k, _CHUNK), jnp.int32),
            pltpu.VMEM((bpw, _L), jnp.float32),
            pltpu.VMEM((bpw, _L), jnp.float32),
            pltpu.VMEM((bpw,), jnp.float32),
            pltpu.VMEM((bpw,), jnp.float32),
            pltpu.SemaphoreType.DMA,
        ],
        compiler_params=pltpu.CompilerParams(
            use_tc_tiling_on_sc=False, needs_layout_passes=False),
    )
    ub, pb = bias_call(uidx3, pidx3, uridx3, pridx3, ubias16, pbias16)

    uodd = ((uidx >> (_PSH - 1)) & 1).reshape(B, 1)
    podd = ((pidx >> (_PSH - 1)) & 1).reshape(B, 1)

    w1u = W1[:, :F].T
    w1p = W1[:, F:].T
    w2t = W2.T
    b1r = b1.reshape(1, H1)
    b2r = b2.reshape(1, H2)
    b3gb = (b3 + global_bias).reshape(1, 1)

    blk = 2048
    out = pl.pallas_call(
        _tc_mlp_body,
        grid=(B // blk,),
        in_specs=[
            pl.BlockSpec((blk, 2 * F), lambda i: (i, 0)),
            pl.BlockSpec((blk, 2 * F), lambda i: (i, 0)),
            pl.BlockSpec((blk, 1), lambda i: (i, 0)),
            pl.BlockSpec((blk, 1), lambda i: (i, 0)),
            pl.BlockSpec((blk, 1), lambda i: (i, 0)),
            pl.BlockSpec((blk, 1), lambda i: (i, 0)),
            pl.BlockSpec((F, H1), lambda i: (0, 0)),
            pl.BlockSpec((F, H1), lambda i: (0, 0)),
            pl.BlockSpec((1, H1), lambda i: (0, 0)),
            pl.BlockSpec((H1, H2), lambda i: (0, 0)),
            pl.BlockSpec((1, H2), lambda i: (0, 0)),
            pl.BlockSpec((1, H2), lambda i: (0, 0)),
            pl.BlockSpec((1, 1), lambda i: (0, 0)),
        ],
        out_specs=pl.BlockSpec((blk, 1), lambda i: (i, 0)),
        out_shape=jax.ShapeDtypeStruct((B, 1), jnp.float32),
    )(u_lines, p_lines, uodd, podd, ub.reshape(B, 1), pb.reshape(B, 1),
      w1u, w1p, b1r, w2t, b2r, W3, b3gb)
    return out[:, 0]


# speedup vs baseline: 1.1430x; 1.1096x over previous
import jax
import jax.numpy as jnp
from jax import lax
from jax.experimental import pallas as pl
from jax.experimental.pallas import tpu as pltpu
from jax.experimental.pallas import tpu_sc as plsc

_NC = 2
_NS = 16
_NW = _NC * _NS
_L = 16
_CHUNK = 128


def _sc_line_gather_body(lidx_hbm, emb2_hbm, out_hbm, lidx_v, lines_v, sem):
    k = lidx_v.shape[0]
    bpw = k * _CHUNK
    wid = lax.axis_index("s") * _NC + lax.axis_index("c")
    base = pl.multiple_of(wid * bpw, bpw)
    pltpu.sync_copy(lidx_hbm.at[wid], lidx_v)
    copies = []
    for j in range(k):
        copies.append(pltpu.async_copy(
            emb2_hbm.at[lidx_v.at[j]],
            lines_v.at[pl.ds(j * _CHUNK, _CHUNK)], sem))
    for c in copies:
        c.wait()
    pltpu.sync_copy(lines_v, out_hbm.at[pl.ds(base, bpw)])


def _sc_bias_gather_body(uidx_hbm, pidx_hbm, uridx_hbm, pridx_hbm,
                         ubias16_hbm, pbias16_hbm, ub_out, pb_out,
                         uidx_v, pidx_v, uridx_v, pridx_v,
                         ubrows_v, pbrows_v, ubvals_v, pbvals_v, sem):
    k = uidx_v.shape[0]
    chunk = uidx_v.shape[1]
    bpw = k * chunk
    wid = lax.axis_index("s") * _NC + lax.axis_index("c")
    base = wid * bpw
    pltpu.sync_copy(uidx_hbm.at[wid], uidx_v)
    pltpu.sync_copy(pidx_hbm.at[wid], pidx_v)
    pltpu.sync_copy(uridx_hbm.at[wid], uridx_v)
    pltpu.sync_copy(pridx_hbm.at[wid], pridx_v)
    copies = []
    for j in range(k):
        sl = pl.ds(j * chunk, chunk)
        copies.append(pltpu.async_copy(
            ubias16_hbm.at[uridx_v.at[j]], ubrows_v.at[sl], sem))
        copies.append(pltpu.async_copy(
            pbias16_hbm.at[pridx_v.at[j]], pbrows_v.at[sl], sem))
    for c in copies:
        c.wait()
    lane_iota = lax.iota(jnp.int32, _L)
    for j in range(k):
        for c in range(chunk // _L):
            off = j * chunk + c * _L
            jvec = off + lane_iota
            usl = uidx_v.at[j][pl.ds(c * _L, _L)] & (_L - 1)
            psl = pidx_v.at[j][pl.ds(c * _L, _L)] & (_L - 1)
            ubvals_v[pl.ds(off, _L)] = plsc.load_gather(ubrows_v, [jvec, usl])
            pbvals_v[pl.ds(off, _L)] = plsc.load_gather(pbrows_v, [jvec, psl])
    pltpu.sync_copy(ubvals_v, ub_out.at[pl.ds(base, bpw)])
    pltpu.sync_copy(pbvals_v, pb_out.at[pl.ds(base, bpw)])


_PBLK = 32768
_PSH = 15


def _tc_pack_body(in_ref, out_ref):
    t = jnp.transpose(in_ref[...])
    h = t.shape[0] // 2
    out_ref[...] = jnp.concatenate([t[:h], t[h:]], axis=1)


def _pack_rows(embT):
    F, N = embT.shape
    nblk = pl.cdiv(N, _PBLK)
    hb = _PBLK // 2
    return pl.pallas_call(
        _tc_pack_body,
        grid=(nblk,),
        in_specs=[pl.BlockSpec((F, _PBLK), lambda i: (0, i))],
        out_specs=pl.BlockSpec((hb, 2 * F), lambda i: (i, 0)),
        out_shape=jax.ShapeDtypeStruct((nblk * hb, 2 * F), jnp.float32),
    )(embT)


def _tc_depad_body(in_ref, out_ref):
    out_ref[...] = in_ref[...][0]


def _depad_bias(bias, blk=131072):
    biasT = bias.T
    N = biasT.shape[1]
    return pl.pallas_call(
        _tc_depad_body,
        grid=(pl.cdiv(N, blk),),
        in_specs=[pl.BlockSpec((1, blk), lambda i: (0, i))],
        out_specs=pl.BlockSpec((blk,), lambda i: (i,)),
        out_shape=jax.ShapeDtypeStruct((N,), jnp.float32),
    )(biasT)


def _tc_mlp_body(ul_ref, pl_ref, uodd_ref, podd_ref, ub_ref, pb_ref,
                 w1u_ref, w1p_ref, b1_ref, w2_ref, b2_ref, w3_ref,
                 b3gb_ref, out_ref):
    f = w1u_ref.shape[0]
    ul = ul_ref[...]
    pll = pl_ref[...]
    u = jnp.where(uodd_ref[...] > 0, ul[:, f:], ul[:, :f])
    p = jnp.where(podd_ref[...] > 0, pll[:, f:], pll[:, :f])
    dot = jnp.sum(u * p, axis=1, keepdims=True)
    h = jnp.dot(u, w1u_ref[...], preferred_element_type=jnp.float32)
    h = h + jnp.dot(p, w1p_ref[...], preferred_element_type=jnp.float32)
    h = jnp.maximum(h + b1_ref[...], 0.0)
    h = jnp.maximum(
        jnp.dot(h, w2_ref[...], preferred_element_type=jnp.float32)
        + b2_ref[...], 0.0)
    mlp = jnp.sum(h * w3_ref[...], axis=1, keepdims=True)
    out_ref[...] = (dot + mlp + ub_ref[...] + pb_ref[...] + b3gb_ref[...])


def kernel(user_idx, prob_idx, user_emb, prob_emb, user_bias, prob_bias,
           global_bias, W1, b1, W2, b2, W3, b3):
    B = user_idx.shape[0]
    F = user_emb.shape[1]
    H1 = W1.shape[0]
    H2 = W2.shape[0]
    bpw = B // _NW
    k = bpw // _CHUNK

    uidx = user_idx.astype(jnp.int32)
    pidx = prob_idx.astype(jnp.int32)
    hb = _PBLK // 2
    ulidx = ((uidx >> _PSH) << (_PSH - 1)) | (uidx & (hb - 1))
    plidx = ((pidx >> _PSH) << (_PSH - 1)) | (pidx & (hb - 1))
    ulidx3 = ulidx.reshape(_NW, k, _CHUNK)
    plidx3 = plidx.reshape(_NW, k, _CHUNK)

    line_call = pl.kernel(
        _sc_line_gather_body,
        out_type=jax.ShapeDtypeStruct((B, 2 * F), jnp.float32),
        mesh=plsc.VectorSubcoreMesh(core_axis_name="c", subcore_axis_name="s"),
        scratch_types=[
            pltpu.VMEM((k, _CHUNK), jnp.int32),
            pltpu.VMEM((bpw, 2 * F), jnp.float32),
            pltpu.SemaphoreType.DMA,
        ],
    )
    pemb2 = _pack_rows(prob_emb.T)
    p_lines = line_call(plidx3, pemb2)
    uemb2 = _pack_rows(user_emb.T)
    u_lines = line_call(ulidx3, uemb2)

    uidx3 = uidx.reshape(_NW, k, _CHUNK)
    pidx3 = pidx.reshape(_NW, k, _CHUNK)
    uridx3 = (uidx >> 4).reshape(_NW, k, _CHUNK)
    pridx3 = (pidx >> 4).reshape(_NW, k, _CHUNK)
    ubias16 = _depad_bias(user_bias).reshape(-1, _L)
    pbias16 = _depad_bias(prob_bias).reshape(-1, _L)

    bias_call = pl.kernel(
        _sc_bias_gather_body,
        out_type=[
            jax.ShapeDtypeStruct((B,), jnp.float32),
            jax.ShapeDtypeStruct((B,), jnp.float32),
        ],
        mesh=plsc.VectorSubcoreMesh(core_axis_name="c", subcore_axis_name="s"),
        scratch_types=[
            pltpu.VMEM((k, _CHUNK), jnp.int32),
            pltpu.VMEM((k, _CHUNK), jnp.int32),
            pltpu.VMEM((k, _CHUNK), jnp.int32),
            pltpu.VMEM((k, _CHUNK), jnp.int32),
            pltpu.VMEM((bpw, _L), jnp.float32),
            pltpu.VMEM((bpw, _L), jnp.float32),
            pltpu.VMEM((bpw,), jnp.float32),
            pltpu.VMEM((bpw,), jnp.float32),
            pltpu.SemaphoreType.DMA,
        ],
        compiler_params=pltpu.CompilerParams(
            use_tc_tiling_on_sc=False, needs_layout_passes=False),
    )
    ub, pb = bias_call(uidx3, pidx3, uridx3, pridx3, ubias16, pbias16)

    uodd = ((uidx >> (_PSH - 1)) & 1).reshape(B, 1)
    podd = ((pidx >> (_PSH - 1)) & 1).reshape(B, 1)

    w1u = W1[:, :F].T
    w1p = W1[:, F:].T
    w2t = W2.T
    b1r = b1.reshape(1, H1)
    b2r = b2.reshape(1, H2)
    b3gb = (b3 + global_bias).reshape(1, 1)

    blk = 2048
    out = pl.pallas_call(
        _tc_mlp_body,
        grid=(B // blk,),
        in_specs=[
            pl.BlockSpec((blk, 2 * F), lambda i: (i, 0)),
            pl.BlockSpec((blk, 2 * F), lambda i: (i, 0)),
            pl.BlockSpec((blk, 1), lambda i: (i, 0)),
            pl.BlockSpec((blk, 1), lambda i: (i, 0)),
            pl.BlockSpec((blk, 1), lambda i: (i, 0)),
            pl.BlockSpec((blk, 1), lambda i: (i, 0)),
            pl.BlockSpec((F, H1), lambda i: (0, 0)),
            pl.BlockSpec((F, H1), lambda i: (0, 0)),
            pl.BlockSpec((1, H1), lambda i: (0, 0)),
            pl.BlockSpec((H1, H2), lambda i: (0, 0)),
            pl.BlockSpec((1, H2), lambda i: (0, 0)),
            pl.BlockSpec((1, H2), lambda i: (0, 0)),
            pl.BlockSpec((1, 1), lambda i: (0, 0)),
        ],
        out_specs=pl.BlockSpec((blk, 1), lambda i: (i, 0)),
        out_shape=jax.ShapeDtypeStruct((B, 1), jnp.float32),
    )(u_lines, p_lines, uodd, podd, ub.reshape(B, 1), pb.reshape(B, 1),
      w1u, w1p, b1r, w2t, b2r, W3, b3gb)
    return out[:, 0]
